# baseline (speedup 1.0000x reference)
import jax
import jax.numpy as jnp
from jax import lax
from jax.experimental import pallas as pl
from jax.experimental.pallas import tpu as pltpu
from jax.experimental.pallas import tpu_sc as plsc

B = 64
N = B * 8192
KSEL = 512
NBLK = N // 16
BUFW = 16384
NVEC = BUFW // 16
NEG = float(jnp.finfo(jnp.float32).min)
NC = 2
NV = KSEL // 16


def _key_u32(f):
    i = lax.bitcast_convert_type(f, jnp.int32)
    m = lax.shift_right_arithmetic(i, 31)
    kk = i ^ (m | jnp.int32(-(2 ** 31)))
    return lax.bitcast_convert_type(kk, jnp.uint32)


def _body(logits_hbm, seg_hbm, outv_hbm, outi_hbm,
          cbuf, kbuf, seg16, pidx, pval, hist, hsum, selv, seli, oeq,
          bnd, bsem):
    wid = lax.axis_index("s") * NC + lax.axis_index("c")
    lanes = lax.iota(jnp.int32, 16)
    lanes256 = lanes * 256
    lanes16 = lanes * 16
    ones16 = jnp.ones((16,), jnp.int32)

    glane = jnp.minimum(lax.div(lanes, jnp.int32(5)), jnp.int32(2))
    jvec = lanes - glane * 5 + 1
    blv = 2 * wid + glane

    def bround(_, c):
        lo0, hi0, lo1, hi1, lo2, hi2 = c

        def pick(a0, a1, a2):
            return jnp.where(glane == 0, a0,
                             jnp.where(glane == 1, a1, a2))

        lov = pick(lo0, lo1, lo2)
        hiv = pick(hi0, hi1, hi2)
        dstep = jnp.maximum(lax.div(hiv - lov, jnp.int32(6)), jnp.int32(1))
        probe = jnp.minimum(lov + jvec * dstep,
                            jnp.maximum(hiv - 1, jnp.int32(0)))
        pidx[...] = probe * 16
        pltpu.async_copy(seg_hbm.at[pidx], pval, bsem).wait()
        less = pval[...] < blv
        cum = plsc.cumsum(jnp.where(less, jnp.int32(1), jnp.int32(0)))
        k0 = cum[4]
        k1 = cum[9] - cum[4]
        k2 = cum[14] - cum[9]

        def upd(lo, hi, k):
            ds_ = jnp.maximum(lax.div(hi - lo, jnp.int32(6)), jnp.int32(1))
            him1 = jnp.maximum(hi - 1, jnp.int32(0))
            nlo = jnp.where(
                k > 0, jnp.minimum(lo + k * ds_, him1) + 1, lo)
            nhi = jnp.where(
                k < 5, jnp.minimum(lo + (k + 1) * ds_, him1), hi)
            pred = lo < hi
            return (jnp.where(pred, nlo, lo), jnp.where(pred, nhi, hi))

        lo0, hi0 = upd(lo0, hi0, k0)
        lo1, hi1 = upd(lo1, hi1, k1)
        lo2, hi2 = upd(lo2, hi2, k2)
        return (lo0, hi0, lo1, hi1, lo2, hi2)

    z = jnp.int32(0)
    nb = jnp.int32(NBLK)
    res = lax.fori_loop(0, 9, bround, (z, nb, z, nb, z, nb))
    bnd[4] = res[0]
    bnd[5] = res[2]
    bnd[6] = res[4]

    def refine(g, _):
        b = 2 * wid + g
        lo = bnd[4 + g]
        blkm1 = jnp.maximum(lo - 1, jnp.int32(0))
        pltpu.sync_copy(
            seg_hbm.at[pl.ds(pl.multiple_of(blkm1 * 16, 16), 16)], seg16)
        cnt = jnp.sum(jnp.where(seg16[...] < b, jnp.int32(1), jnp.int32(0)))
        bnd[g] = jnp.where(lo == 0, jnp.int32(0), blkm1 * 16 + cnt)
        return 0

    lax.fori_loop(0, 3, refine, 0)

    def group(g, _):
        row = 2 * wid + g
        s = bnd[g]
        e = bnd[g + 1]
        s16 = s & jnp.int32(-16)
        base = pl.multiple_of(jnp.minimum(s16, jnp.int32(N - BUFW)), 8)
        fits = (e - base) <= BUFW
        jlo = lax.shift_right_arithmetic(s - base, jnp.int32(4))
        nvt = lax.div(e - base + jnp.int32(15), jnp.int32(16))
        nch = lax.div(e - s16 + jnp.int32(BUFW - 1), jnp.int32(BUFW))

        def zero_hist():
            zv = jnp.zeros((16,), jnp.int32)

            def z(i, _):
                for u in range(8):
                    hist[pl.ds(i * 128 + u * 16, 16)] = zv
                return 0
            lax.fori_loop(0, 32, z, 0)

        def scan_buckets(R):
            def lred(cb, _):
                b16 = cb * 256 + lanes16
                acc = plsc.load_gather(hist, [b16])
                for l in range(1, 16):
                    acc = acc + plsc.load_gather(hist, [b16 + l])
                hsum[pl.ds(cb * 16, 16)] = acc
                return 0
            lax.fori_loop(0, 16, lred, 0)

            def coarse(i, c):
                running, found, cbx, runb = c
                cb = 15 - i
                v = lax.rev(hsum[pl.ds(cb * 16, 16)], (0,))
                tb = plsc.cumsum(v)[15]
                nr = running + tb
                crossed = (found == 0) & (nr >= R)
                cbx = jnp.where(crossed, cb, cbx)
                runb = jnp.where(crossed, running, runb)
                return (nr, found | jnp.where(crossed, 1, 0), cbx, runb)

            _, found, cbx, runb = lax.fori_loop(
                0, 16, coarse,
                (jnp.int32(0), jnp.int32(0), jnp.int32(0), jnp.int32(0)))

            v = hsum[pl.ds(cbx * 16, 16)]
            rv = lax.rev(v, (0,))
            cum = plsc.cumsum(rv) + runb
            crossed = cum >= R
            cumex = cum - rv
            pc = plsc.all_reduce_population_count(crossed)[0]
            fl = jnp.int32(16) - pc
            beta = cbx * 16 + 15 - fl
            runbef = jnp.min(jnp.where(crossed, cumex, jnp.int32(2 ** 30)))
            beta = jnp.where(found == 0, jnp.int32(0), beta)
            rn = jnp.where(found == 0, R, R - runbef)
            return beta, rn

        def zero_sel():
            def z(i, _):
                selv[pl.ds(i * 16, 16)] = jnp.full((16,), NEG, jnp.float32)
                seli[pl.ds(i * 16, 16)] = jnp.zeros((16,), jnp.int32)
                return 0
            lax.fori_loop(0, NV, z, 0)

        @pl.when(fits)
        def _():
            pltpu.sync_copy(logits_hbm.at[pl.ds(base, BUFW)],
                            cbuf.at[pl.ds(0, BUFW)])
            up4 = jlo + ((nvt - jlo + jnp.int32(3)) & jnp.int32(-4))

            zk = jnp.zeros((16,), jnp.uint32)
            for u in range(3):
                kbuf[pl.ds((nvt + u) * 16, 16)] = zk

            T = jnp.uint32(0)
            R = jnp.int32(KSEL)
            for lvl in range(4):
                shift = 24 - 8 * lvl
                zero_hist()

                if lvl == 0:
                    @pl.loop(jlo, up4, step=4)
                    def _h(j):
                        for u in range(4):
                            ju = j + u
                            kr = _key_u32(cbuf[pl.ds(ju * 16, 16)])
                            gidx = base + ju * 16 + lanes
                            valid = (gidx >= s) & (gidx < e)
                            ku = jnp.where(valid, kr, jnp.uint32(0))
                            kbuf[pl.ds(ju * 16, 16)] = ku
                            bucket = (ku >> jnp.uint32(24)).astype(jnp.int32)
                            plsc.addupdate_scatter(
                                hist, [bucket * 16 + lanes], ones16)
                else:
                    hs = jnp.uint32(shift + 8)
                    Ths = T >> hs

                    @pl.loop(jlo, up4, step=4)
                    def _h(j, shift=shift, hs=hs, Ths=Ths):
                        for u in range(4):
                            ku = kbuf[pl.ds((j + u) * 16, 16)]
                            pm = (ku >> hs) == Ths
                            bucket = ((ku >> jnp.uint32(shift))
                                      & jnp.uint32(255)).astype(jnp.int32)
                            plsc.addupdate_scatter(
                                hist, [bucket * 16 + lanes], ones16, mask=pm)

                beta, R = scan_buckets(R)
                T = T | (beta.astype(jnp.uint32) << jnp.uint32(shift))

            quota = R
            zero_sel()
            z16 = jnp.zeros((16,), jnp.int32)

            @pl.loop(jlo, up4, step=2, init_carry=(z16, z16))
            def _compact(j, carry):
                for u in range(2):
                    outoff, eqcnt = carry
                    ju = j + u
                    ku = kbuf[pl.ds(ju * 16, 16)]
                    gt = ku > T
                    eq = (ku == T) & (ku != jnp.uint32(0))
                    ceq = plsc.cumsum(
                        jnp.where(eq, jnp.int32(1), jnp.int32(0)))
                    keep = gt | (eq & (ceq + eqcnt <= quota))
                    ki = jnp.where(keep, jnp.int32(1), jnp.int32(0))
                    ck = plsc.cumsum(ki)
                    pos = outoff + ck - ki
                    f = cbuf[pl.ds(ju * 16, 16)]
                    gidx = base + ju * 16 + lanes
                    plsc.store_scatter(selv, [pos], f, mask=keep)
                    plsc.store_scatter(seli, [pos], gidx, mask=keep)
                    carry = (outoff + jnp.full((16,), ck[15], jnp.int32),
                             eqcnt + jnp.full((16,), ceq[15], jnp.int32))
                return carry

        @pl.when(jnp.logical_not(fits))
        def _():
            def hist_vec(f, gidx, T, shift, lvl):
                ku = _key_u32(f)
                valid = (gidx >= s) & (gidx < e)
                if lvl > 0:
                    hs = jnp.uint32(shift + 8)
                    valid = valid & ((ku >> hs) == (T >> hs))
                bucket = ((ku >> jnp.uint32(shift))
                          & jnp.uint32(255)).astype(jnp.int32)
                plsc.addupdate_scatter(
                    hist, [bucket * 16 + lanes], ones16, mask=valid)

            def compact_vec(f, gidx, T, quota, carry):
                outoff, eqcnt = carry
                ku = _key_u32(f)
                valid = (gidx >= s) & (gidx < e)
                gt = valid & (ku > T)
                eq = valid & (ku == T)
                ceq = plsc.cumsum(jnp.where(eq, jnp.int32(1), jnp.int32(0)))
                keep = gt | (eq & (ceq + eqcnt <= quota))
                ki = jnp.where(keep, jnp.int32(1), jnp.int32(0))
                ck = plsc.cumsum(ki)
                pos = outoff + ck - ki
                plsc.store_scatter(selv, [pos], f, mask=keep)
                plsc.store_scatter(seli, [pos], gidx, mask=keep)
                return (outoff + jnp.full((16,), ck[15], jnp.int32),
                        eqcnt + jnp.full((16,), ceq[15], jnp.int32))

            def stream(pb, carry=None):
                if carry is not None:
                    oeq[pl.ds(0, 16)] = carry[0]
                    oeq[pl.ds(16, 16)] = carry[1]

                @pl.loop(jnp.int32(0), nch)
                def _chunks(c):
                    cb = pl.multiple_of(
                        jnp.minimum(s16 + c * BUFW, jnp.int32(N - BUFW)), 8)
                    jhi = jnp.minimum(
                        lax.div(e - cb + jnp.int32(15), jnp.int32(16)),
                        jnp.int32(NVEC))
                    pltpu.sync_copy(logits_hbm.at[pl.ds(cb, BUFW)],
                                    cbuf.at[pl.ds(0, BUFW)])
                    wlo = jnp.maximum(s, s16 + c * BUFW)
                    whi = jnp.minimum(e, s16 + (c + 1) * BUFW)

                    if carry is None:
                        @pl.loop(jnp.int32(0), jhi)
                        def _vecs(j):
                            f = cbuf[pl.ds(j * 16, 16)]
                            gidx = cb + j * 16 + lanes
                            ok = (gidx >= wlo) & (gidx < whi)
                            fm = jnp.where(ok, f, jnp.float32(NEG))
                            gm = jnp.where(ok, gidx, jnp.int32(-1))
                            pb(fm, gm)
                    else:
                        cc = (oeq[pl.ds(0, 16)], oeq[pl.ds(16, 16)])

                        @pl.loop(jnp.int32(0), jhi, init_carry=cc)
                        def _vecs(j, c2):
                            f = cbuf[pl.ds(j * 16, 16)]
                            gidx = cb + j * 16 + lanes
                            ok = (gidx >= wlo) & (gidx < whi)
                            fm = jnp.where(ok, f, jnp.float32(NEG))
                            gm = jnp.where(ok, gidx, jnp.int32(-1))
                            return pb(fm, gm, c2)

                        oeq[pl.ds(0, 16)] = _vecs[0]
                        oeq[pl.ds(16, 16)] = _vecs[1]

            T = jnp.uint32(0)
            R = jnp.int32(KSEL)
            for lvl in range(4):
                shift = 24 - 8 * lvl
                zero_hist()
                stream(lambda f, gidx, T=T, shift=shift, lvl=lvl:
                       hist_vec(f, gidx, T, shift, lvl))
                beta, R = scan_buckets(R)
                T = T | (beta.astype(jnp.uint32) << jnp.uint32(shift))
            quota = R
            zero_sel()
            z16 = jnp.zeros((16,), jnp.int32)
            stream(lambda f, gidx, carry, T=T, quota=quota:
                   compact_vec(f, gidx, T, quota, carry), (z16, z16))

        def vsort_at(i):
            kk, vv = plsc.sort_key_val(
                selv[pl.ds(i * 16, 16)], seli[pl.ds(i * 16, 16)],
                descending=True)
            selv[pl.ds(i * 16, 16)] = kk
            seli[pl.ds(i * 16, 16)] = vv

        def ce(i, j):
            ka = selv[pl.ds(i * 16, 16)]
            kb = selv[pl.ds(j * 16, 16)]
            va = seli[pl.ds(i * 16, 16)]
            vb = seli[pl.ds(j * 16, 16)]
            c = ka >= kb
            selv[pl.ds(i * 16, 16)] = jnp.where(c, ka, kb)
            selv[pl.ds(j * 16, 16)] = jnp.where(c, kb, ka)
            seli[pl.ds(i * 16, 16)] = jnp.where(c, va, vb)
            seli[pl.ds(j * 16, 16)] = jnp.where(c, vb, va)

        def rev_at(i):
            selv[pl.ds(i * 16, 16)] = lax.rev(selv[pl.ds(i * 16, 16)], (0,))
            seli[pl.ds(i * 16, 16)] = lax.rev(seli[pl.ds(i * 16, 16)], (0,))

        def rev_swap(i, j):
            ka = selv[pl.ds(i * 16, 16)]
            kb = selv[pl.ds(j * 16, 16)]
            va = seli[pl.ds(i * 16, 16)]
            vb = seli[pl.ds(j * 16, 16)]
            selv[pl.ds(i * 16, 16)] = lax.rev(kb, (0,))
            selv[pl.ds(j * 16, 16)] = lax.rev(ka, (0,))
            seli[pl.ds(i * 16, 16)] = lax.rev(vb, (0,))
            seli[pl.ds(j * 16, 16)] = lax.rev(va, (0,))

        for i in range(NV):
            vsort_at(i)
        for w in (1, 2, 4, 8, 16):
            for basev in range(0, NV, 2 * w):
                bstart = basev + w
                if w == 1:
                    rev_at(bstart)
                else:
                    for r in range(w // 2):
                        rev_swap(bstart + r, bstart + w - 1 - r)
                d = w
                while d >= 1:
                    for blk in range(basev, basev + 2 * w, 2 * d):
                        for t in range(d):
                            ce(blk + t, blk + t + d)
                    d //= 2
                for i in range(basev, basev + 2 * w):
                    vsort_at(i)

        pltpu.sync_copy(selv, outv_hbm.at[row])
        pltpu.sync_copy(seli, outi_hbm.at[row])
        return 0

    lax.fori_loop(0, 2, group, 0)


def kernel(logits, segment_ids, k):
    mesh = plsc.VectorSubcoreMesh(core_axis_name="c", subcore_axis_name="s")
    fn = pl.kernel(
        _body,
        out_type=[
            jax.ShapeDtypeStruct((B, KSEL), jnp.float32),
            jax.ShapeDtypeStruct((B, KSEL), jnp.int32),
        ],
        mesh=mesh,
        compiler_params=pltpu.CompilerParams(needs_layout_passes=False),
        scratch_types=[
            pltpu.VMEM((BUFW + 48,), jnp.float32),
            pltpu.VMEM((BUFW + 48,), jnp.uint32),
            pltpu.VMEM((16,), jnp.int32),
            pltpu.VMEM((16,), jnp.int32),
            pltpu.VMEM((16,), jnp.int32),
            pltpu.VMEM((4096,), jnp.int32),
            pltpu.VMEM((256,), jnp.int32),
            pltpu.VMEM((KSEL,), jnp.float32),
            pltpu.VMEM((KSEL,), jnp.int32),
            pltpu.VMEM((32,), jnp.int32),
            pltpu.SMEM((8,), jnp.int32),
            pltpu.SemaphoreType.DMA,
        ],
    )
    vals, idx = fn(logits, segment_ids)
    idx = idx + (jnp.asarray(k, jnp.int32) - jnp.int32(KSEL))
    return vals, idx


# speedup vs baseline: 1.1472x; 1.1472x over previous
import jax
import jax.numpy as jnp
from jax import lax
from jax.experimental import pallas as pl
from jax.experimental.pallas import tpu as pltpu
from jax.experimental.pallas import tpu_sc as plsc

B = 64
N = B * 8192
KSEL = 512
NBLK = N // 16
BUFW = 16384
NVEC = BUFW // 16
NEG = float(jnp.finfo(jnp.float32).min)
NC = 2
NV = KSEL // 16


def _key_u32(f):
    i = lax.bitcast_convert_type(f, jnp.int32)
    m = lax.shift_right_arithmetic(i, 31)
    kk = i ^ (m | jnp.int32(-(2 ** 31)))
    return lax.bitcast_convert_type(kk, jnp.uint32)


def _body(logits_hbm, seg_hbm, outv_hbm, outi_hbm,
          cbuf, kbuf, seg16, pidx, pval, candv, candi, hist, hsum, selv,
          seli, oeq, bnd, bsem):
    wid = lax.axis_index("s") * NC + lax.axis_index("c")
    lanes = lax.iota(jnp.int32, 16)
    lanes256 = lanes * 256
    ones16 = jnp.ones((16,), jnp.int32)

    glane = jnp.minimum(lax.div(lanes, jnp.int32(5)), jnp.int32(2))
    jvec = lanes - glane * 5 + 1
    blv = 2 * wid + glane

    def bround(_, c):
        lo0, hi0, lo1, hi1, lo2, hi2 = c

        def pick(a0, a1, a2):
            return jnp.where(glane == 0, a0,
                             jnp.where(glane == 1, a1, a2))

        lov = pick(lo0, lo1, lo2)
        hiv = pick(hi0, hi1, hi2)
        dstep = jnp.maximum(lax.div(hiv - lov, jnp.int32(6)), jnp.int32(1))
        probe = jnp.minimum(lov + jvec * dstep,
                            jnp.maximum(hiv - 1, jnp.int32(0)))
        pidx[...] = probe * 16
        pltpu.async_copy(seg_hbm.at[pidx], pval, bsem).wait()
        less = pval[...] < blv
        cum = plsc.cumsum(jnp.where(less, jnp.int32(1), jnp.int32(0)))
        k0 = cum[4]
        k1 = cum[9] - cum[4]
        k2 = cum[14] - cum[9]

        def upd(lo, hi, k):
            ds_ = jnp.maximum(lax.div(hi - lo, jnp.int32(6)), jnp.int32(1))
            him1 = jnp.maximum(hi - 1, jnp.int32(0))
            nlo = jnp.where(
                k > 0, jnp.minimum(lo + k * ds_, him1) + 1, lo)
            nhi = jnp.where(
                k < 5, jnp.minimum(lo + (k + 1) * ds_, him1), hi)
            pred = lo < hi
            return (jnp.where(pred, nlo, lo), jnp.where(pred, nhi, hi))

        lo0, hi0 = upd(lo0, hi0, k0)
        lo1, hi1 = upd(lo1, hi1, k1)
        lo2, hi2 = upd(lo2, hi2, k2)
        return (lo0, hi0, lo1, hi1, lo2, hi2)

    z = jnp.int32(0)
    nb = jnp.int32(NBLK)
    res = lax.fori_loop(0, 9, bround, (z, nb, z, nb, z, nb))
    bnd[4] = res[0]
    bnd[5] = res[2]
    bnd[6] = res[4]

    def refine(g, _):
        b = 2 * wid + g
        lo = bnd[4 + g]
        blkm1 = jnp.maximum(lo - 1, jnp.int32(0))
        pltpu.sync_copy(
            seg_hbm.at[pl.ds(pl.multiple_of(blkm1 * 16, 16), 16)], seg16)
        cnt = jnp.sum(jnp.where(seg16[...] < b, jnp.int32(1), jnp.int32(0)))
        bnd[g] = jnp.where(lo == 0, jnp.int32(0), blkm1 * 16 + cnt)
        return 0

    lax.fori_loop(0, 3, refine, 0)

    def group(g, _):
        row = 2 * wid + g
        s = bnd[g]
        e = bnd[g + 1]
        s16 = s & jnp.int32(-16)
        base = pl.multiple_of(jnp.minimum(s16, jnp.int32(N - BUFW)), 8)
        fits = (e - base) <= BUFW
        jlo = lax.shift_right_arithmetic(s - base, jnp.int32(4))
        nvt = lax.div(e - base + jnp.int32(15), jnp.int32(16))
        nch = lax.div(e - s16 + jnp.int32(BUFW - 1), jnp.int32(BUFW))

        def zero_hist():
            zv = jnp.zeros((16,), jnp.int32)

            def z(i, _):
                for u in range(8):
                    hist[pl.ds(i * 128 + u * 16, 16)] = zv
                return 0
            lax.fori_loop(0, 32, z, 0)

        def scan_buckets(R):
            def lred(cb, _):
                acc = hist[pl.ds(cb * 16, 16)]
                for l in range(1, 16):
                    acc = acc + hist[pl.ds(l * 256 + cb * 16, 16)]
                hsum[pl.ds(cb * 16, 16)] = acc
                return 0
            lax.fori_loop(0, 16, lred, 0)

            def coarse(i, c):
                running, found, cbx, runb = c
                cb = 15 - i
                v = lax.rev(hsum[pl.ds(cb * 16, 16)], (0,))
                tb = plsc.cumsum(v)[15]
                nr = running + tb
                crossed = (found == 0) & (nr >= R)
                cbx = jnp.where(crossed, cb, cbx)
                runb = jnp.where(crossed, running, runb)
                return (nr, found | jnp.where(crossed, 1, 0), cbx, runb)

            _, found, cbx, runb = lax.fori_loop(
                0, 16, coarse,
                (jnp.int32(0), jnp.int32(0), jnp.int32(0), jnp.int32(0)))

            v = hsum[pl.ds(cbx * 16, 16)]
            rv = lax.rev(v, (0,))
            cum = plsc.cumsum(rv) + runb
            crossed = cum >= R
            cumex = cum - rv
            pc = plsc.all_reduce_population_count(crossed)[0]
            fl = jnp.int32(16) - pc
            beta = cbx * 16 + 15 - fl
            runbef = jnp.min(jnp.where(crossed, cumex, jnp.int32(2 ** 30)))
            beta = jnp.where(found == 0, jnp.int32(0), beta)
            rn = jnp.where(found == 0, R, R - runbef)
            return beta, rn

        def zero_sel():
            def z(i, _):
                selv[pl.ds(i * 16, 16)] = jnp.full((16,), NEG, jnp.float32)
                seli[pl.ds(i * 16, 16)] = jnp.zeros((16,), jnp.int32)
                return 0
            lax.fori_loop(0, NV, z, 0)

        @pl.when(fits)
        def _():
            pltpu.sync_copy(logits_hbm.at[pl.ds(base, BUFW)],
                            cbuf.at[pl.ds(0, BUFW)])
            up4 = jlo + ((nvt - jlo + jnp.int32(3)) & jnp.int32(-4))

            zk = jnp.zeros((16,), jnp.uint32)
            for u in range(3):
                kbuf[pl.ds((nvt + u) * 16, 16)] = zk

            T = jnp.uint32(0)
            R = jnp.int32(KSEL)
            for lvl in range(3):
                shift = 24 - 8 * lvl
                zero_hist()

                if lvl == 0:
                    @pl.loop(jlo, up4, step=4)
                    def _h(j):
                        for u in range(4):
                            ju = j + u
                            kr = _key_u32(cbuf[pl.ds(ju * 16, 16)])
                            gidx = base + ju * 16 + lanes
                            valid = (gidx >= s) & (gidx < e)
                            ku = jnp.where(valid, kr, jnp.uint32(0))
                            kbuf[pl.ds(ju * 16, 16)] = ku
                            bucket = (ku >> jnp.uint32(24)).astype(jnp.int32)
                            plsc.addupdate_scatter(
                                hist, [lanes256 + bucket], ones16)
                else:
                    hs = jnp.uint32(shift + 8)
                    Ths = T >> hs

                    @pl.loop(jlo, up4, step=4)
                    def _h(j, shift=shift, hs=hs, Ths=Ths):
                        for u in range(4):
                            ku = kbuf[pl.ds((j + u) * 16, 16)]
                            pm = (ku >> hs) == Ths
                            bucket = ((ku >> jnp.uint32(shift))
                                      & jnp.uint32(255)).astype(jnp.int32)
                            plsc.addupdate_scatter(
                                hist, [lanes256 + bucket], ones16, mask=pm)

                beta, R = scan_buckets(R)
                T = T | (beta.astype(jnp.uint32) << jnp.uint32(shift))

            T24 = T >> jnp.uint32(8)
            zero_sel()
            candv[...] = jnp.full((16,), NEG, jnp.float32)
            candi[...] = jnp.zeros((16,), jnp.int32)
            z16 = jnp.zeros((16,), jnp.int32)

            @pl.loop(jlo, up4, step=2, init_carry=(z16, z16))
            def _compact(j, carry):
                for u in range(2):
                    outoff, candoff = carry
                    ju = j + u
                    ku = kbuf[pl.ds(ju * 16, 16)]
                    k24 = ku >> jnp.uint32(8)
                    gt = k24 > T24
                    eq = (k24 == T24) & (ku != jnp.uint32(0))
                    gi = jnp.where(gt, jnp.int32(1), jnp.int32(0))
                    cg = plsc.cumsum(gi)
                    pos = outoff + cg - gi
                    f = cbuf[pl.ds(ju * 16, 16)]
                    gidx = base + ju * 16 + lanes
                    plsc.store_scatter(selv, [pos], f, mask=gt)
                    plsc.store_scatter(seli, [pos], gidx, mask=gt)
                    ei = jnp.where(eq, jnp.int32(1), jnp.int32(0))
                    ce_ = plsc.cumsum(ei)
                    cpos = candoff + ce_ - ei
                    cm = eq & (cpos < 16)
                    plsc.store_scatter(candv, [cpos], f, mask=cm)
                    plsc.store_scatter(candi, [cpos], gidx, mask=cm)
                    carry = (outoff + jnp.full((16,), cg[15], jnp.int32),
                             candoff + jnp.full((16,), ce_[15], jnp.int32))
                return carry

            ngt, ncand = _compact
            ngt0 = ngt[0]
            ncand0 = ncand[0]

            @pl.when(ncand0 <= 16)
            def _():
                sv, si = plsc.sort_key_val(
                    candv[...], candi[...], descending=True)
                cm = (lanes < R) & ((ngt0 + lanes) < KSEL)
                plsc.store_scatter(selv, [ngt0 + lanes], sv, mask=cm)
                plsc.store_scatter(seli, [ngt0 + lanes], si, mask=cm)

            @pl.when(ncand0 > 16)
            def _():
                zero_hist()
                Ths = T >> jnp.uint32(8)

                @pl.loop(jlo, up4, step=4)
                def _h4(j):
                    for u in range(4):
                        ku = kbuf[pl.ds((j + u) * 16, 16)]
                        pm = (ku >> jnp.uint32(8)) == Ths
                        bucket = (ku & jnp.uint32(255)).astype(jnp.int32)
                        plsc.addupdate_scatter(
                            hist, [lanes256 + bucket], ones16, mask=pm)

                beta4, R4 = scan_buckets(R)
                T4 = T | beta4.astype(jnp.uint32)
                quota = R4
                zero_sel()

                @pl.loop(jlo, up4, step=2, init_carry=(z16, z16))
                def _compact4(j, carry):
                    for u in range(2):
                        outoff, eqcnt = carry
                        ju = j + u
                        ku = kbuf[pl.ds(ju * 16, 16)]
                        gt = ku > T4
                        eq = (ku == T4) & (ku != jnp.uint32(0))
                        ceq = plsc.cumsum(
                            jnp.where(eq, jnp.int32(1), jnp.int32(0)))
                        keep = gt | (eq & (ceq + eqcnt <= quota))
                        ki = jnp.where(keep, jnp.int32(1), jnp.int32(0))
                        ck = plsc.cumsum(ki)
                        pos = outoff + ck - ki
                        f = cbuf[pl.ds(ju * 16, 16)]
                        gidx = base + ju * 16 + lanes
                        plsc.store_scatter(selv, [pos], f, mask=keep)
                        plsc.store_scatter(seli, [pos], gidx, mask=keep)
                        carry = (
                            outoff + jnp.full((16,), ck[15], jnp.int32),
                            eqcnt + jnp.full((16,), ceq[15], jnp.int32))
                    return carry

        @pl.when(jnp.logical_not(fits))
        def _():
            def hist_vec(f, gidx, T, shift, lvl):
                ku = _key_u32(f)
                valid = (gidx >= s) & (gidx < e)
                if lvl > 0:
                    hs = jnp.uint32(shift + 8)
                    valid = valid & ((ku >> hs) == (T >> hs))
                bucket = ((ku >> jnp.uint32(shift))
                          & jnp.uint32(255)).astype(jnp.int32)
                plsc.addupdate_scatter(
                    hist, [lanes256 + bucket], ones16, mask=valid)

            def compact_vec(f, gidx, T, quota, carry):
                outoff, eqcnt = carry
                ku = _key_u32(f)
                valid = (gidx >= s) & (gidx < e)
                gt = valid & (ku > T)
                eq = valid & (ku == T)
                ceq = plsc.cumsum(jnp.where(eq, jnp.int32(1), jnp.int32(0)))
                keep = gt | (eq & (ceq + eqcnt <= quota))
                ki = jnp.where(keep, jnp.int32(1), jnp.int32(0))
                ck = plsc.cumsum(ki)
                pos = outoff + ck - ki
                plsc.store_scatter(selv, [pos], f, mask=keep)
                plsc.store_scatter(seli, [pos], gidx, mask=keep)
                return (outoff + jnp.full((16,), ck[15], jnp.int32),
                        eqcnt + jnp.full((16,), ceq[15], jnp.int32))

            def stream(pb, carry=None):
                if carry is not None:
                    oeq[pl.ds(0, 16)] = carry[0]
                    oeq[pl.ds(16, 16)] = carry[1]

                @pl.loop(jnp.int32(0), nch)
                def _chunks(c):
                    cb = pl.multiple_of(
                        jnp.minimum(s16 + c * BUFW, jnp.int32(N - BUFW)), 8)
                    jhi = jnp.minimum(
                        lax.div(e - cb + jnp.int32(15), jnp.int32(16)),
                        jnp.int32(NVEC))
                    pltpu.sync_copy(logits_hbm.at[pl.ds(cb, BUFW)],
                                    cbuf.at[pl.ds(0, BUFW)])
                    wlo = jnp.maximum(s, s16 + c * BUFW)
                    whi = jnp.minimum(e, s16 + (c + 1) * BUFW)

                    if carry is None:
                        @pl.loop(jnp.int32(0), jhi)
                        def _vecs(j):
                            f = cbuf[pl.ds(j * 16, 16)]
                            gidx = cb + j * 16 + lanes
                            ok = (gidx >= wlo) & (gidx < whi)
                            fm = jnp.where(ok, f, jnp.float32(NEG))
                            gm = jnp.where(ok, gidx, jnp.int32(-1))
                            pb(fm, gm)
                    else:
                        cc = (oeq[pl.ds(0, 16)], oeq[pl.ds(16, 16)])

                        @pl.loop(jnp.int32(0), jhi, init_carry=cc)
                        def _vecs(j, c2):
                            f = cbuf[pl.ds(j * 16, 16)]
                            gidx = cb + j * 16 + lanes
                            ok = (gidx >= wlo) & (gidx < whi)
                            fm = jnp.where(ok, f, jnp.float32(NEG))
                            gm = jnp.where(ok, gidx, jnp.int32(-1))
                            return pb(fm, gm, c2)

                        oeq[pl.ds(0, 16)] = _vecs[0]
                        oeq[pl.ds(16, 16)] = _vecs[1]

            T = jnp.uint32(0)
            R = jnp.int32(KSEL)
            for lvl in range(4):
                shift = 24 - 8 * lvl
                zero_hist()
                stream(lambda f, gidx, T=T, shift=shift, lvl=lvl:
                       hist_vec(f, gidx, T, shift, lvl))
                beta, R = scan_buckets(R)
                T = T | (beta.astype(jnp.uint32) << jnp.uint32(shift))
            quota = R
            zero_sel()
            z16 = jnp.zeros((16,), jnp.int32)
            stream(lambda f, gidx, carry, T=T, quota=quota:
                   compact_vec(f, gidx, T, quota, carry), (z16, z16))

        def vsort_at(i):
            kk, vv = plsc.sort_key_val(
                selv[pl.ds(i * 16, 16)], seli[pl.ds(i * 16, 16)],
                descending=True)
            selv[pl.ds(i * 16, 16)] = kk
            seli[pl.ds(i * 16, 16)] = vv

        def ce(i, j):
            ka = selv[pl.ds(i * 16, 16)]
            kb = selv[pl.ds(j * 16, 16)]
            va = seli[pl.ds(i * 16, 16)]
            vb = seli[pl.ds(j * 16, 16)]
            c = ka >= kb
            selv[pl.ds(i * 16, 16)] = jnp.where(c, ka, kb)
            selv[pl.ds(j * 16, 16)] = jnp.where(c, kb, ka)
            seli[pl.ds(i * 16, 16)] = jnp.where(c, va, vb)
            seli[pl.ds(j * 16, 16)] = jnp.where(c, vb, va)

        def rev_at(i):
            selv[pl.ds(i * 16, 16)] = lax.rev(selv[pl.ds(i * 16, 16)], (0,))
            seli[pl.ds(i * 16, 16)] = lax.rev(seli[pl.ds(i * 16, 16)], (0,))

        def rev_swap(i, j):
            ka = selv[pl.ds(i * 16, 16)]
            kb = selv[pl.ds(j * 16, 16)]
            va = seli[pl.ds(i * 16, 16)]
            vb = seli[pl.ds(j * 16, 16)]
            selv[pl.ds(i * 16, 16)] = lax.rev(kb, (0,))
            selv[pl.ds(j * 16, 16)] = lax.rev(ka, (0,))
            seli[pl.ds(i * 16, 16)] = lax.rev(vb, (0,))
            seli[pl.ds(j * 16, 16)] = lax.rev(va, (0,))

        for i in range(NV):
            vsort_at(i)
        for w in (1, 2, 4, 8, 16):
            for basev in range(0, NV, 2 * w):
                bstart = basev + w
                if w == 1:
                    rev_at(bstart)
                else:
                    for r in range(w // 2):
                        rev_swap(bstart + r, bstart + w - 1 - r)
                d = w
                while d >= 1:
                    for blk in range(basev, basev + 2 * w, 2 * d):
                        for t in range(d):
                            ce(blk + t, blk + t + d)
                    d //= 2
                for i in range(basev, basev + 2 * w):
                    vsort_at(i)

        pltpu.sync_copy(selv, outv_hbm.at[row])
        pltpu.sync_copy(seli, outi_hbm.at[row])
        return 0

    lax.fori_loop(0, 2, group, 0)


def kernel(logits, segment_ids, k):
    mesh = plsc.VectorSubcoreMesh(core_axis_name="c", subcore_axis_name="s")
    fn = pl.kernel(
        _body,
        out_type=[
            jax.ShapeDtypeStruct((B, KSEL), jnp.float32),
            jax.ShapeDtypeStruct((B, KSEL), jnp.int32),
        ],
        mesh=mesh,
        compiler_params=pltpu.CompilerParams(needs_layout_passes=False),
        scratch_types=[
            pltpu.VMEM((BUFW + 48,), jnp.float32),
            pltpu.VMEM((BUFW + 48,), jnp.uint32),
            pltpu.VMEM((16,), jnp.int32),
            pltpu.VMEM((16,), jnp.int32),
            pltpu.VMEM((16,), jnp.int32),
            pltpu.VMEM((16,), jnp.float32),
            pltpu.VMEM((16,), jnp.int32),
            pltpu.VMEM((4096,), jnp.int32),
            pltpu.VMEM((256,), jnp.int32),
            pltpu.VMEM((KSEL,), jnp.float32),
            pltpu.VMEM((KSEL,), jnp.int32),
            pltpu.VMEM((32,), jnp.int32),
            pltpu.SMEM((8,), jnp.int32),
            pltpu.SemaphoreType.DMA,
        ],
    )
    vals, idx = fn(logits, segment_ids)
    idx = idx + (jnp.asarray(k, jnp.int32) - jnp.int32(KSEL))
    return vals, idx
